# baseline (speedup 1.0000x reference)
import jax
import jax.numpy as jnp
from jax import lax
from jax.experimental import pallas as pl
from jax.experimental.pallas import tpu as pltpu
from jax.experimental.pallas import tpu_sc as plsc

NC = 2
NS = 16
NW = NC * NS

D = 128
B = 16384 * 200
ROWS_PER_WORKER = B // NW
CHUNK = 128
ITERS = ROWS_PER_WORKER // CHUNK


def _sc_body(pos_hbm, table_hbm, out_hbm, idx_v, rows_v, sem):
  wid = lax.axis_index("s") * NC + lax.axis_index("c")
  worker_base = wid * ROWS_PER_WORKER

  def body(g, carry):
    base = pl.multiple_of(worker_base + g * CHUNK, CHUNK)
    pltpu.sync_copy(pos_hbm.at[pl.ds(base, CHUNK)], idx_v)
    pltpu.async_copy(table_hbm.at[idx_v], rows_v, sem).wait()
    pltpu.sync_copy(rows_v, out_hbm.at[pl.ds(base, CHUNK)])
    return carry

  lax.fori_loop(0, ITERS, body, 0, unroll=False)


@jax.jit
def _embed(pos_flat, table):
  mesh = plsc.VectorSubcoreMesh(core_axis_name="c", subcore_axis_name="s")
  return pl.kernel(
      _sc_body,
      out_type=jax.ShapeDtypeStruct((B, D), jnp.float32),
      mesh=mesh,
      scratch_types=[
          pltpu.VMEM((CHUNK,), jnp.int32),
          pltpu.VMEM((CHUNK, D), jnp.float32),
          pltpu.SemaphoreType.DMA,
      ],
  )(pos_flat, table)


def kernel(pos, seg_embd_weight):
  pos_flat = pos.reshape(B).astype(jnp.int32)
  out = _embed(pos_flat, seg_embd_weight)
  return out.reshape(pos.shape + (D,))


# speedup vs baseline: 107.3015x; 107.3015x over previous
import jax
import jax.numpy as jnp
from jax import lax
from jax.experimental import pallas as pl
from jax.experimental.pallas import tpu as pltpu
from jax.experimental.pallas import tpu_sc as plsc

NC = 2
NS = 16
NW = NC * NS
L = 16

D = 128
B = 16384 * 200
ROWS_PER_WORKER = B // NW
CHUNK = 400
ITERS = ROWS_PER_WORKER // CHUNK
NKB = D // L


def _sc_body(pos_hbm, table_hbm, out_hbm, tab_v, idx0, idx1, rows0, rows1,
             si0, si1, so0, so1):
  si = (si0, si1)
  so = (so0, so1)
  idx_b = (idx0, idx1)
  rows_b = (rows0, rows1)
  wid = lax.axis_index("s") * NC + lax.axis_index("c")
  wbase = wid * ROWS_PER_WORKER

  pltpu.sync_copy(table_hbm, tab_v)
  w0 = [tab_v[0, pl.ds(k * L, L)] for k in range(NKB)]
  dw = [tab_v[1, pl.ds(k * L, L)] - w0[k] for k in range(NKB)]

  def idx_start(b, i):
    src = pos_hbm.at[pl.ds((wbase + i * CHUNK), CHUNK)]
    pltpu.async_copy(src, idx_b[b], si[b])

  def idx_wait(b):
    pltpu.make_async_copy(pos_hbm.at[pl.ds(0, CHUNK)], idx_b[b],
                          si[b]).wait()

  def out_start(b, i):
    dst = out_hbm.at[pl.ds((wbase + i * CHUNK) * D, CHUNK * D)]
    pltpu.async_copy(rows_b[b], dst, so[b])

  def out_wait(b):
    pltpu.make_async_copy(rows_b[b], out_hbm.at[pl.ds(0, CHUNK * D)],
                          so[b]).wait()

  def compute(b):
    idx_ref = idx_b[b]
    rows_ref = rows_b[b]

    def group_body(g, carry):
      p16 = idx_ref[pl.ds(pl.multiple_of(g * L, L), L)].astype(jnp.float32)
      gbase = pl.multiple_of(g * L * D, L * D)
      for r in range(L):
        pf = jnp.full((L,), p16[r])
        for k in range(NKB):
          rows_ref[pl.ds(gbase + r * D + k * L, L)] = w0[k] + pf * dw[k]
      return carry

    lax.fori_loop(0, CHUNK // L, group_body, 0, unroll=False)

  idx_start(0, 0)
  idx_start(1, 1)
  for b in range(2):
    idx_wait(b)
    compute(b)
    out_start(b, b)
    idx_start(b, b + 2)

  def main(k2, carry):
    for j in range(2):
      i = 2 + k2 * 2 + j
      b = j
      idx_wait(b)
      out_wait(b)
      compute(b)
      out_start(b, i)
      pl.when(i + 2 < ITERS)(lambda: idx_start(b, i + 2))
    return carry

  lax.fori_loop(0, (ITERS - 2) // 2, main, 0, unroll=False)

  for b in range(2):
    out_wait(b)


@jax.jit
def _embed(pos_flat, table):
  mesh = plsc.VectorSubcoreMesh(core_axis_name="c", subcore_axis_name="s")
  return pl.kernel(
      _sc_body,
      out_type=jax.ShapeDtypeStruct((B * D,), jnp.float32),
      mesh=mesh,
      scratch_types=[
          pltpu.VMEM((2, D), jnp.float32),
          pltpu.VMEM((CHUNK,), jnp.int32),
          pltpu.VMEM((CHUNK,), jnp.int32),
          pltpu.VMEM((CHUNK * D,), jnp.float32),
          pltpu.VMEM((CHUNK * D,), jnp.float32),
          pltpu.SemaphoreType.DMA,
          pltpu.SemaphoreType.DMA,
          pltpu.SemaphoreType.DMA,
          pltpu.SemaphoreType.DMA,
      ],
  )(pos_flat, table)


def kernel(pos, seg_embd_weight):
  pos_flat = pos.reshape(B).astype(jnp.int32)
  out = _embed(pos_flat, seg_embd_weight)
  return out.reshape(pos.shape + (D,))
